# baseline (speedup 1.0000x reference)
import functools

import jax
import jax.numpy as jnp
from jax import lax
from jax.experimental import pallas as pl
from jax.experimental.pallas import tpu as pltpu
from jax.experimental.pallas import tpu_sc as plsc

N, Q, K = 10000, 128, 10
BM1 = 2000

SC_ROWS = 2560
TC_ROWS = N - SC_ROWS
BM2 = 240

NW = 32
RPT = SC_ROWS // NW
GD = 8
HG = 4
CHM = 1280
CHT = 1152
NCH = 8
NGRP = RPT // GD
NUNIT = NGRP * NCH


def _stage1_body(z_ref, wtT_ref, bt_ref, lnw_ref, lnb_ref, woT_ref, bo_ref,
                 p_ref, pbf_ref, s_ref, x_ref, colsum_ref, ztp_ref):
    step = pl.program_id(0)
    nsteps = pl.num_programs(0)

    z = z_ref[...]
    m = jnp.dot(z, wtT_ref[...], preferred_element_type=jnp.float32)
    m = m + bt_ref[...]
    mu = jnp.mean(m, axis=1, keepdims=True)
    var = jnp.mean((m - mu) * (m - mu), axis=1, keepdims=True)
    mn = (m - mu) / jnp.sqrt(var + 1e-5) * lnw_ref[...] + lnb_ref[...]
    h = jnp.where(mn >= 0, mn, 0.2 * mn)
    ol = jnp.dot(h, woT_ref[...], preferred_element_type=jnp.float32)
    ol = ol + bo_ref[...]
    olmax = jnp.max(ol, axis=1, keepdims=True)
    e = jnp.exp(ol - olmax)
    p = e / jnp.sum(e, axis=1, keepdims=True)
    p_ref[...] = p
    pbf_ref[...] = p.astype(jnp.bfloat16)

    pmax = jnp.max(p, axis=1, keepdims=True)
    lane = lax.broadcasted_iota(jnp.int32, p.shape, 1)
    s_ref[...] = jnp.min(jnp.where(p == pmax, lane, 127), axis=1,
                         keepdims=True)

    @pl.when(step == 0)
    def _init():
        colsum_ref[...] = jnp.zeros_like(colsum_ref)
        ztp_ref[...] = jnp.zeros_like(ztp_ref)

    colsum_ref[...] += jnp.sum(p, axis=0, keepdims=True)
    ztp_ref[...] += lax.dot_general(z, p, (((0,), (0,)), ((), ())),
                                    preferred_element_type=jnp.float32)

    @pl.when(step == nsteps - 1)
    def _fin():
        cs = colsum_ref[...]
        lane1 = lax.broadcasted_iota(jnp.int32, cs.shape, 1)
        d = jnp.where(lane1 < K, 1.0 / cs + 1e-8, 0.0)
        x_ref[...] = (ztp_ref[...] * d).T


def _stage2_body(a_ref, pbf_ref, pblk_ref, at_ref, acc_ref):
    step = pl.program_id(0)
    nsteps = pl.num_programs(0)

    a_bf = a_ref[...].astype(jnp.bfloat16)
    ap = jnp.dot(a_bf, pbf_ref[...], preferred_element_type=jnp.float32)

    @pl.when(step == 0)
    def _init():
        acc_ref[...] = jnp.zeros_like(acc_ref)

    acc_ref[...] += lax.dot_general(pblk_ref[...], ap,
                                    (((0,), (0,)), ((), ())),
                                    preferred_element_type=jnp.float32)

    @pl.when(step == nsteps - 1)
    def _fin():
        at_ref[...] = acc_ref[...]


NTAIL = (N - (NCH - 1) * CHM) // 16


def _sc_body(a_hbm, atail_hbm, pt_hbm, out_hbm,
             pcols, abuf, accA, accB, sem0, sem1, semoA, semoB, semp):
    sid = lax.axis_index("s")
    cid = lax.axis_index("c")
    wid = sid * 2 + cid
    row0 = TC_ROWS + wid * RPT
    row0t = wid * RPT
    pltpu.async_copy(pt_hbm.at[pl.ds(0, K * N)], pcols, semp).wait()

    def _dma(g, cj, b, sem, start):
        rs = pl.multiple_of(row0 + g * GD, GD)
        if cj < NCH - 1:
            cp = pltpu.make_async_copy(
                a_hbm.at[pl.ds(rs, GD), pl.ds(cj * CHM, CHM)],
                abuf.at[b], sem)
        else:
            rst = pl.multiple_of(row0t + g * GD, GD)
            cp = pltpu.make_async_copy(
                atail_hbm.at[pl.ds(rst, GD), :], abuf.at[b, :, 0:CHT], sem)
        cp.start() if start else cp.wait()

    def _out_copy(g, accbuf, semo):
        return pltpu.make_async_copy(
            accbuf,
            out_hbm.at[pl.ds((row0t + g * GD) * K * 16, GD * K * 16)],
            semo)

    def _group(g, accbuf, semo, last):
        @pl.when(g >= 2)
        def _():
            _out_copy(g - 2, accbuf, semo).wait()

        for cj in range(NCH):
            b = cj % 2
            _dma(g, cj, b, (sem0, sem1)[b], False)
            if cj < NCH - 1:
                _dma(g, cj + 1, 1 - b, (sem0, sem1)[1 - b], True)
            else:

                @pl.when(jnp.logical_not(last))
                def _():
                    _dma(g + 1, 0, 1 - b, (sem0, sem1)[1 - b], True)

            iters = CHM // 16 if cj < NCH - 1 else NTAIL
            for h in range(GD // HG):
                if cj == 0:
                    accs = [jnp.zeros((16,), jnp.float32)
                            for _ in range(HG * K)]
                else:
                    accs = [accbuf[pl.ds(((h * HG + r) * K + c) * 16, 16)]
                            for r in range(HG) for c in range(K)]

                def body(jj, carry, b=b, h=h, cj=cj):
                    accs = list(carry)
                    off = jj * 16
                    pvs = [pcols[pl.ds(c * N + cj * CHM + off, 16)]
                           for c in range(K)]
                    for r in range(HG):
                        ar = abuf[b, h * HG + r, pl.ds(off, 16)]
                        for c in range(K):
                            accs[r * K + c] = accs[r * K + c] + ar * pvs[c]
                    return tuple(accs)

                accs = list(lax.fori_loop(0, iters, body, tuple(accs)))

                for r in range(HG):
                    for c in range(K):
                        accbuf[pl.ds(((h * HG + r) * K + c) * 16, 16)] = (
                            accs[r * K + c])

        _out_copy(g, accbuf, semo).start()

    _dma(0, 0, 0, sem0, True)

    def gpair_body(t, _):
        g0 = t * 2
        _group(g0, accA, semoA, False)
        _group(g0 + 1, accB, semoB, g0 + 1 >= NGRP - 1)
        return 0

    lax.fori_loop(0, NGRP // 2, gpair_body, 0)

    _out_copy(NGRP - 2, accA, semoA).wait()
    _out_copy(NGRP - 1, accB, semoB).wait()


_sc_call = functools.partial(
    pl.kernel,
    mesh=plsc.VectorSubcoreMesh(core_axis_name="c", subcore_axis_name="s"),
    out_type=jax.ShapeDtypeStruct((SC_ROWS * K * 16,), jnp.float32),
    scratch_types=[
        pltpu.VMEM((K * N,), jnp.float32),
        pltpu.VMEM((2, GD, CHM), jnp.float32),
        pltpu.VMEM((GD * K * 16,), jnp.float32),
        pltpu.VMEM((GD * K * 16,), jnp.float32),
        pltpu.SemaphoreType.DMA,
        pltpu.SemaphoreType.DMA,
        pltpu.SemaphoreType.DMA,
        pltpu.SemaphoreType.DMA,
        pltpu.SemaphoreType.DMA,
    ],
)(_sc_body)


def _stage2b_body(b_ref, f_ref, psc_ref, at_ref):
    ap = jnp.dot(b_ref[...], f_ref[...], preferred_element_type=jnp.float32)
    at_ref[...] = lax.dot_general(psc_ref[...], ap, (((0,), (0,)), ((), ())),
                                  preferred_element_type=jnp.float32)


def kernel(Z, A, W_t, b_t, ln_w, ln_b, W_o, b_o):
    wtT = W_t.T
    bt = b_t.reshape(1, Q)
    lnw = ln_w.reshape(1, Q)
    lnb = ln_b.reshape(1, Q)
    woT = jnp.zeros((Q, 128), jnp.float32).at[:, :K].set(W_o.T)
    bo = jnp.full((1, 128), -1e30, jnp.float32).at[0, :K].set(b_o)

    grid1 = N // BM1
    p_pad, p_bf, s2d, x_full = pl.pallas_call(
        _stage1_body,
        grid=(grid1,),
        in_specs=[
            pl.BlockSpec((BM1, Q), lambda i: (i, 0)),
            pl.BlockSpec((Q, Q), lambda i: (0, 0)),
            pl.BlockSpec((1, Q), lambda i: (0, 0)),
            pl.BlockSpec((1, Q), lambda i: (0, 0)),
            pl.BlockSpec((1, Q), lambda i: (0, 0)),
            pl.BlockSpec((Q, 128), lambda i: (0, 0)),
            pl.BlockSpec((1, 128), lambda i: (0, 0)),
        ],
        out_specs=[
            pl.BlockSpec((BM1, 128), lambda i: (i, 0)),
            pl.BlockSpec((BM1, 128), lambda i: (i, 0)),
            pl.BlockSpec((BM1, 1), lambda i: (i, 0)),
            pl.BlockSpec((128, 128), lambda i: (0, 0)),
        ],
        out_shape=[
            jax.ShapeDtypeStruct((N, 128), jnp.float32),
            jax.ShapeDtypeStruct((N, 128), jnp.bfloat16),
            jax.ShapeDtypeStruct((N, 1), jnp.int32),
            jax.ShapeDtypeStruct((128, 128), jnp.float32),
        ],
        scratch_shapes=[
            pltpu.VMEM((1, 128), jnp.float32),
            pltpu.VMEM((128, 128), jnp.float32),
        ],
    )(Z, wtT, bt, lnw, lnb, woT, bo)

    pt = p_pad[:, :K].T.reshape(-1)
    atail = jnp.pad(A[TC_ROWS:, (NCH - 1) * CHM:],
                    ((0, 0), (0, CHT - (N - (NCH - 1) * CHM))))

    b_sc = _sc_call(A, atail, pt).reshape(SC_ROWS, K * 16)

    fold = (lax.broadcasted_iota(jnp.int32, (K * 16, 128), 0) // 16
            == lax.broadcasted_iota(jnp.int32, (K * 16, 128), 1)
            ).astype(jnp.float32)

    at_sc = pl.pallas_call(
        _stage2b_body,
        grid=(1,),
        in_specs=[
            pl.BlockSpec((SC_ROWS, K * 16), lambda i: (0, 0)),
            pl.BlockSpec((K * 16, 128), lambda i: (0, 0)),
            pl.BlockSpec((SC_ROWS, 128), lambda i: (0, 0)),
        ],
        out_specs=pl.BlockSpec((128, 128), lambda i: (0, 0)),
        out_shape=jax.ShapeDtypeStruct((128, 128), jnp.float32),
    )(b_sc, fold, p_pad[TC_ROWS:])

    at_tc = pl.pallas_call(
        _stage2_body,
        grid=(TC_ROWS // BM2,),
        in_specs=[
            pl.BlockSpec((BM2, N), lambda i: (i, 0)),
            pl.BlockSpec((N, 128), lambda i: (0, 0)),
            pl.BlockSpec((BM2, 128), lambda i: (i, 0)),
        ],
        out_specs=pl.BlockSpec((128, 128), lambda i: (0, 0)),
        out_shape=jax.ShapeDtypeStruct((128, 128), jnp.float32),
        scratch_shapes=[pltpu.VMEM((128, 128), jnp.float32)],
    )(A, p_bf, p_pad)

    X_tilde = x_full[:K, :]
    A_tilde = at_tc[:K, :K] + at_sc[:K, :K]
    P = p_pad[:, :K]
    S = s2d[:, 0]
    return X_tilde, A_tilde, P, S


# speedup vs baseline: 1.5517x; 1.5517x over previous
import functools

import jax
import jax.numpy as jnp
from jax import lax
from jax.experimental import pallas as pl
from jax.experimental.pallas import tpu as pltpu
from jax.experimental.pallas import tpu_sc as plsc

N, Q, K = 10000, 128, 10
BM1 = 2000

SC_ROWS = 768
TC_ROWS = N - SC_ROWS
BM2 = 400
G2 = 24

NW = 32
RPT = SC_ROWS // NW
GD = 8
HG = 4
CHM = 1280
CHT = 1024
NCH = 8
NTAIL = CHT // 16
NGRP = RPT // GD
SC_COLS = (NCH - 1) * CHM + CHT


def _stage1_body(z_ref, wtT_ref, bt_ref, lnw_ref, lnb_ref, woT_ref, bo_ref,
                 p_ref, pbf_ref, s_ref, x_ref, colsum_ref, ztp_ref):
    step = pl.program_id(0)
    nsteps = pl.num_programs(0)

    z = z_ref[...]
    m = jnp.dot(z, wtT_ref[...], preferred_element_type=jnp.float32)
    m = m + bt_ref[...]
    mu = jnp.mean(m, axis=1, keepdims=True)
    var = jnp.mean((m - mu) * (m - mu), axis=1, keepdims=True)
    mn = (m - mu) / jnp.sqrt(var + 1e-5) * lnw_ref[...] + lnb_ref[...]
    h = jnp.where(mn >= 0, mn, 0.2 * mn)
    ol = jnp.dot(h, woT_ref[...], preferred_element_type=jnp.float32)
    ol = ol + bo_ref[...]
    olmax = jnp.max(ol, axis=1, keepdims=True)
    e = jnp.exp(ol - olmax)
    p = e / jnp.sum(e, axis=1, keepdims=True)
    p_ref[...] = p[:, :K]
    pbf_ref[...] = p.astype(jnp.bfloat16)

    pmax = jnp.max(p, axis=1, keepdims=True)
    lane = lax.broadcasted_iota(jnp.int32, p.shape, 1)
    s_ref[...] = jnp.min(jnp.where(p == pmax, lane, 127), axis=1,
                         keepdims=True)

    @pl.when(step == 0)
    def _init():
        colsum_ref[...] = jnp.zeros_like(colsum_ref)
        ztp_ref[...] = jnp.zeros_like(ztp_ref)

    colsum_ref[...] += jnp.sum(p, axis=0, keepdims=True)
    ztp_ref[...] += lax.dot_general(z, p, (((0,), (0,)), ((), ())),
                                    preferred_element_type=jnp.float32)

    @pl.when(step == nsteps - 1)
    def _fin():
        cs = colsum_ref[...]
        lane1 = lax.broadcasted_iota(jnp.int32, cs.shape, 1)
        d = jnp.where(lane1 < K, 1.0 / cs + 1e-8, 0.0)
        x_ref[...] = (ztp_ref[...] * d).T[:K, :]


def _stage2_body(a_ref, pbf_ref, pblk_ref, at_ref, acc_ref):
    step = pl.program_id(0)
    nsteps = pl.num_programs(0)

    a_bf = a_ref[...].astype(jnp.bfloat16)
    ap = jnp.dot(a_bf, pbf_ref[...], preferred_element_type=jnp.float32)

    grow = lax.broadcasted_iota(jnp.int32, (BM2, 128), 0) + step * BM2
    pblk = jnp.where(grow < TC_ROWS, pblk_ref[...],
                     jnp.bfloat16(0)).astype(jnp.float32)

    @pl.when(step == 0)
    def _init():
        acc_ref[...] = jnp.zeros_like(acc_ref)

    acc_ref[...] += lax.dot_general(pblk, ap, (((0,), (0,)), ((), ())),
                                    preferred_element_type=jnp.float32)

    @pl.when(step == nsteps - 1)
    def _fin():
        at_ref[...] = acc_ref[...]


def _sc_body(a_hbm, pt_hbm, out_hbm,
             pcols, abuf, accA, accB, sem0, sem1, semoA, semoB, semp):
    sid = lax.axis_index("s")
    cid = lax.axis_index("c")
    wid = sid * 2 + cid
    row0 = TC_ROWS + wid * RPT
    row0t = wid * RPT
    pltpu.async_copy(pt_hbm.at[pl.ds(0, K * N)], pcols, semp).wait()

    def _dma(g, cj, b, sem, start):
        rs = pl.multiple_of(row0 + g * GD, GD)
        if cj < NCH - 1:
            cp = pltpu.make_async_copy(
                a_hbm.at[pl.ds(rs, GD), pl.ds(cj * CHM, CHM)],
                abuf.at[b], sem)
        else:
            cp = pltpu.make_async_copy(
                a_hbm.at[pl.ds(rs, GD), pl.ds((NCH - 1) * CHM, CHT)],
                abuf.at[b, :, 0:CHT], sem)
        cp.start() if start else cp.wait()

    def _out_copy(g, accbuf, semo):
        return pltpu.make_async_copy(
            accbuf,
            out_hbm.at[pl.ds((row0t + g * GD) * K * 16, GD * K * 16)],
            semo)

    def _group(g, accbuf, semo, last):
        for cj in range(NCH):
            b = cj % 2
            _dma(g, cj, b, (sem0, sem1)[b], False)
            if cj < NCH - 1:
                _dma(g, cj + 1, 1 - b, (sem0, sem1)[1 - b], True)
            elif not last:
                _dma(g + 1, 0, 1 - b, (sem0, sem1)[1 - b], True)

            iters = CHM // 16 if cj < NCH - 1 else NTAIL
            for h in range(GD // HG):
                if cj == 0:
                    accs = [jnp.zeros((16,), jnp.float32)
                            for _ in range(HG * K)]
                else:
                    accs = [accbuf[pl.ds(((h * HG + r) * K + c) * 16, 16)]
                            for r in range(HG) for c in range(K)]

                def body(jj, carry, b=b, h=h, cj=cj):
                    accs = list(carry)
                    off = jj * 16
                    pvs = [pcols[pl.ds(c * N + cj * CHM + off, 16)]
                           for c in range(K)]
                    for r in range(HG):
                        ar = abuf[b, h * HG + r, pl.ds(off, 16)]
                        for c in range(K):
                            accs[r * K + c] = accs[r * K + c] + ar * pvs[c]
                    return tuple(accs)

                accs = list(lax.fori_loop(0, iters, body, tuple(accs)))

                for r in range(HG):
                    for c in range(K):
                        accbuf[pl.ds(((h * HG + r) * K + c) * 16, 16)] = (
                            accs[r * K + c])

        _out_copy(g, accbuf, semo).start()

    _dma(0, 0, 0, sem0, True)
    _group(0, accA, semoA, False)
    _group(1, accB, semoB, False)
    _group(2, accA, semoA, True)

    _out_copy(0, accA, semoA).wait()
    _out_copy(1, accB, semoB).wait()
    _out_copy(2, accA, semoA).wait()


_sc_call = functools.partial(
    pl.kernel,
    mesh=plsc.VectorSubcoreMesh(core_axis_name="c", subcore_axis_name="s"),
    out_type=jax.ShapeDtypeStruct((SC_ROWS * K * 16,), jnp.float32),
    scratch_types=[
        pltpu.VMEM((K * N,), jnp.float32),
        pltpu.VMEM((2, GD, CHM), jnp.float32),
        pltpu.VMEM((GD * K * 16,), jnp.float32),
        pltpu.VMEM((GD * K * 16,), jnp.float32),
        pltpu.SemaphoreType.DMA,
        pltpu.SemaphoreType.DMA,
        pltpu.SemaphoreType.DMA,
        pltpu.SemaphoreType.DMA,
        pltpu.SemaphoreType.DMA,
    ],
)(_sc_body)


def _stage2b_body(b_ref, f_ref, strip_ref, pstrip_ref, psc_ref, attc_ref,
                  at_ref):
    ap = jnp.dot(b_ref[...], f_ref[...], preferred_element_type=jnp.float32)
    ap = ap + jnp.dot(strip_ref[...], pstrip_ref[...],
                      preferred_element_type=jnp.float32)
    at = lax.dot_general(psc_ref[...], ap, (((0,), (0,)), ((), ())),
                         preferred_element_type=jnp.float32)
    at_ref[...] = (attc_ref[...] + at)[:K, :K]


def kernel(Z, A, W_t, b_t, ln_w, ln_b, W_o, b_o):
    wtT = W_t.T
    bt = b_t.reshape(1, Q)
    lnw = ln_w.reshape(1, Q)
    lnb = ln_b.reshape(1, Q)
    woT = jnp.zeros((Q, 128), jnp.float32).at[:, :K].set(W_o.T)
    bo = jnp.full((1, 128), -1e30, jnp.float32).at[0, :K].set(b_o)

    grid1 = N // BM1
    P, p_bf, s2d, X_tilde = pl.pallas_call(
        _stage1_body,
        grid=(grid1,),
        in_specs=[
            pl.BlockSpec((BM1, Q), lambda i: (i, 0)),
            pl.BlockSpec((Q, Q), lambda i: (0, 0)),
            pl.BlockSpec((1, Q), lambda i: (0, 0)),
            pl.BlockSpec((1, Q), lambda i: (0, 0)),
            pl.BlockSpec((1, Q), lambda i: (0, 0)),
            pl.BlockSpec((Q, 128), lambda i: (0, 0)),
            pl.BlockSpec((1, 128), lambda i: (0, 0)),
        ],
        out_specs=[
            pl.BlockSpec((BM1, K), lambda i: (i, 0)),
            pl.BlockSpec((BM1, 128), lambda i: (i, 0)),
            pl.BlockSpec((BM1, 1), lambda i: (i, 0)),
            pl.BlockSpec((K, Q), lambda i: (0, 0)),
        ],
        out_shape=[
            jax.ShapeDtypeStruct((N, K), jnp.float32),
            jax.ShapeDtypeStruct((N, 128), jnp.bfloat16),
            jax.ShapeDtypeStruct((N, 1), jnp.int32),
            jax.ShapeDtypeStruct((K, Q), jnp.float32),
        ],
        scratch_shapes=[
            pltpu.VMEM((1, 128), jnp.float32),
            pltpu.VMEM((128, 128), jnp.float32),
        ],
    )(Z, wtT, bt, lnw, lnb, woT, bo)

    pt = p_bf[:, :K].astype(jnp.float32).T.reshape(-1)

    b_sc = _sc_call(A, pt).reshape(SC_ROWS, K * 16)

    at_tc = pl.pallas_call(
        _stage2_body,
        grid=(G2,),
        in_specs=[
            pl.BlockSpec((BM2, N), lambda i: (i, 0)),
            pl.BlockSpec((N, 128), lambda i: (0, 0)),
            pl.BlockSpec((BM2, 128), lambda i: (i, 0)),
        ],
        out_specs=pl.BlockSpec((128, 128), lambda i: (0, 0)),
        out_shape=jax.ShapeDtypeStruct((128, 128), jnp.float32),
        scratch_shapes=[pltpu.VMEM((128, 128), jnp.float32)],
    )(A, p_bf, p_bf)

    fold = (lax.broadcasted_iota(jnp.int32, (K * 16, 128), 0) // 16
            == lax.broadcasted_iota(jnp.int32, (K * 16, 128), 1)
            ).astype(jnp.float32)

    A_tilde = pl.pallas_call(
        _stage2b_body,
        grid=(1,),
        in_specs=[
            pl.BlockSpec((SC_ROWS, K * 16), lambda i: (0, 0)),
            pl.BlockSpec((K * 16, 128), lambda i: (0, 0)),
            pl.BlockSpec((SC_ROWS, N - SC_COLS), lambda i: (0, 0)),
            pl.BlockSpec((N - SC_COLS, 128), lambda i: (0, 0)),
            pl.BlockSpec((SC_ROWS, 128), lambda i: (0, 0)),
            pl.BlockSpec((128, 128), lambda i: (0, 0)),
        ],
        out_specs=pl.BlockSpec((K, K), lambda i: (0, 0)),
        out_shape=jax.ShapeDtypeStruct((K, K), jnp.float32),
    )(b_sc, fold, A[TC_ROWS:, SC_COLS:],
      p_bf[SC_COLS:].astype(jnp.float32), p_bf[TC_ROWS:].astype(jnp.float32),
      at_tc)

    S = s2d[:, 0]
    return X_tilde, A_tilde, P, S


# speedup vs baseline: 1.6631x; 1.0718x over previous
import functools

import jax
import jax.numpy as jnp
from jax import lax
from jax.experimental import pallas as pl
from jax.experimental.pallas import tpu as pltpu
from jax.experimental.pallas import tpu_sc as plsc

N, Q, K = 10000, 128, 10
BM1 = 2000

SC_ROWS = 512
TC_ROWS = N - SC_ROWS
BM2 = 400
G2 = 24

NW = 32
RPT = SC_ROWS // NW
GD = 8
HG = 4
CHM = 1280
CHT = 1024
NCH = 8
NTAIL = CHT // 16
NGRP = RPT // GD
SC_COLS = (NCH - 1) * CHM + CHT


def _stage1_body(z_ref, wtT_ref, bt_ref, lnw_ref, lnb_ref, woT_ref, bo_ref,
                 p_ref, pbf_ref, s_ref, x_ref, colsum_ref, ztp_ref):
    step = pl.program_id(0)
    nsteps = pl.num_programs(0)

    z = z_ref[...]
    m = jnp.dot(z, wtT_ref[...], preferred_element_type=jnp.float32)
    m = m + bt_ref[...]
    mu = jnp.mean(m, axis=1, keepdims=True)
    var = jnp.mean((m - mu) * (m - mu), axis=1, keepdims=True)
    mn = (m - mu) / jnp.sqrt(var + 1e-5) * lnw_ref[...] + lnb_ref[...]
    h = jnp.where(mn >= 0, mn, 0.2 * mn)
    ol = jnp.dot(h, woT_ref[...], preferred_element_type=jnp.float32)
    ol = ol + bo_ref[...]
    olmax = jnp.max(ol, axis=1, keepdims=True)
    e = jnp.exp(ol - olmax)
    p = e / jnp.sum(e, axis=1, keepdims=True)
    p_ref[...] = p[:, :K]
    pbf_ref[...] = p.astype(jnp.bfloat16)

    pmax = jnp.max(p, axis=1, keepdims=True)
    lane = lax.broadcasted_iota(jnp.int32, p.shape, 1)
    s_ref[...] = jnp.min(jnp.where(p == pmax, lane, 127), axis=1,
                         keepdims=True)

    @pl.when(step == 0)
    def _init():
        colsum_ref[...] = jnp.zeros_like(colsum_ref)
        ztp_ref[...] = jnp.zeros_like(ztp_ref)

    colsum_ref[...] += jnp.sum(p, axis=0, keepdims=True)
    ztp_ref[...] += lax.dot_general(z, p, (((0,), (0,)), ((), ())),
                                    preferred_element_type=jnp.float32)

    @pl.when(step == nsteps - 1)
    def _fin():
        cs = colsum_ref[...]
        lane1 = lax.broadcasted_iota(jnp.int32, cs.shape, 1)
        d = jnp.where(lane1 < K, 1.0 / cs + 1e-8, 0.0)
        x_ref[...] = (ztp_ref[...] * d).T[:K, :]


def _stage2_body(a_ref, pbf_ref, pblk_ref, at_ref, acc_ref):
    step = pl.program_id(0)
    nsteps = pl.num_programs(0)

    a_bf = a_ref[...].astype(jnp.bfloat16)
    ap = jnp.dot(a_bf, pbf_ref[...], preferred_element_type=jnp.float32)

    grow = lax.broadcasted_iota(jnp.int32, (BM2, 128), 0) + step * BM2
    pblk = jnp.where(grow < TC_ROWS, pblk_ref[...],
                     jnp.bfloat16(0)).astype(jnp.float32)

    @pl.when(step == 0)
    def _init():
        acc_ref[...] = jnp.zeros_like(acc_ref)

    acc_ref[...] += lax.dot_general(pblk, ap, (((0,), (0,)), ((), ())),
                                    preferred_element_type=jnp.float32)

    @pl.when(step == nsteps - 1)
    def _fin():
        at_ref[...] = acc_ref[...]


def _sc_body(a_hbm, pt_hbm, out_hbm,
             ptspm, pcols, abuf, accA, accB, sem0, sem1, semoA, semoB,
             semp):
    sid = lax.axis_index("s")
    cid = lax.axis_index("c")
    wid = sid * 2 + cid
    row0 = TC_ROWS + wid * RPT
    row0t = wid * RPT
    @pl.when(sid == 0)
    def _():
        pltpu.async_copy(pt_hbm.at[pl.ds(0, K * N)], ptspm, semp).wait()

    plsc.subcore_barrier()
    pltpu.async_copy(ptspm.at[pl.ds(0, K * N)], pcols, semp).wait()

    def _dma(g, cj, b, sem, start):
        rs = pl.multiple_of(row0 + g * GD, GD)
        if cj < NCH - 1:
            cp = pltpu.make_async_copy(
                a_hbm.at[pl.ds(rs, GD), pl.ds(cj * CHM, CHM)],
                abuf.at[b], sem)
        else:
            cp = pltpu.make_async_copy(
                a_hbm.at[pl.ds(rs, GD), pl.ds((NCH - 1) * CHM, CHT)],
                abuf.at[b, :, 0:CHT], sem)
        cp.start() if start else cp.wait()

    def _out_copy(g, accbuf, semo):
        return pltpu.make_async_copy(
            accbuf,
            out_hbm.at[pl.ds((row0t + g * GD) * K * 16, GD * K * 16)],
            semo)

    def _group(g, accbuf, semo, last):
        for cj in range(NCH):
            b = cj % 2
            _dma(g, cj, b, (sem0, sem1)[b], False)
            if cj < NCH - 1:
                _dma(g, cj + 1, 1 - b, (sem0, sem1)[1 - b], True)
            elif not last:
                _dma(g + 1, 0, 1 - b, (sem0, sem1)[1 - b], True)

            iters = CHM // 16 if cj < NCH - 1 else NTAIL
            for h in range(GD // HG):
                if cj == 0:
                    accs = [jnp.zeros((16,), jnp.float32)
                            for _ in range(HG * K)]
                else:
                    accs = [accbuf[pl.ds(((h * HG + r) * K + c) * 16, 16)]
                            for r in range(HG) for c in range(K)]

                def body(jj, carry, b=b, h=h, cj=cj):
                    accs = list(carry)
                    off = jj * 16
                    pvs = [pcols[pl.ds(c * N + cj * CHM + off, 16)]
                           for c in range(K)]
                    for r in range(HG):
                        ar = abuf[b, h * HG + r, pl.ds(off, 16)]
                        for c in range(K):
                            accs[r * K + c] = accs[r * K + c] + ar * pvs[c]
                    return tuple(accs)

                accs = list(lax.fori_loop(0, iters, body, tuple(accs)))

                for r in range(HG):
                    for c in range(K):
                        accbuf[pl.ds(((h * HG + r) * K + c) * 16, 16)] = (
                            accs[r * K + c])

        _out_copy(g, accbuf, semo).start()

    _dma(0, 0, 0, sem0, True)
    _group(0, accA, semoA, False)
    _group(1, accB, semoB, True)

    _out_copy(0, accA, semoA).wait()
    _out_copy(1, accB, semoB).wait()


_sc_call = functools.partial(
    pl.kernel,
    mesh=plsc.VectorSubcoreMesh(core_axis_name="c", subcore_axis_name="s"),
    out_type=jax.ShapeDtypeStruct((SC_ROWS * K * 16,), jnp.float32),
    scratch_types=[
        pltpu.VMEM_SHARED((K * N,), jnp.float32),
        pltpu.VMEM((K * N,), jnp.float32),
        pltpu.VMEM((2, GD, CHM), jnp.float32),
        pltpu.VMEM((GD * K * 16,), jnp.float32),
        pltpu.VMEM((GD * K * 16,), jnp.float32),
        pltpu.SemaphoreType.DMA,
        pltpu.SemaphoreType.DMA,
        pltpu.SemaphoreType.DMA,
        pltpu.SemaphoreType.DMA,
        pltpu.SemaphoreType.DMA,
    ],
)(_sc_body)


def _stage2b_body(b_ref, f_ref, strip_ref, pstrip_ref, psc_ref, attc_ref,
                  at_ref):
    ap = jnp.dot(b_ref[...], f_ref[...], preferred_element_type=jnp.float32)
    ap = ap + jnp.dot(strip_ref[...], pstrip_ref[...],
                      preferred_element_type=jnp.float32)
    at = lax.dot_general(psc_ref[...], ap, (((0,), (0,)), ((), ())),
                         preferred_element_type=jnp.float32)
    at_ref[...] = (attc_ref[...] + at)[:K, :K]


def kernel(Z, A, W_t, b_t, ln_w, ln_b, W_o, b_o):
    wtT = W_t.T
    bt = b_t.reshape(1, Q)
    lnw = ln_w.reshape(1, Q)
    lnb = ln_b.reshape(1, Q)
    woT = jnp.zeros((Q, 128), jnp.float32).at[:, :K].set(W_o.T)
    bo = jnp.full((1, 128), -1e30, jnp.float32).at[0, :K].set(b_o)

    grid1 = N // BM1
    P, p_bf, s2d, X_tilde = pl.pallas_call(
        _stage1_body,
        grid=(grid1,),
        in_specs=[
            pl.BlockSpec((BM1, Q), lambda i: (i, 0)),
            pl.BlockSpec((Q, Q), lambda i: (0, 0)),
            pl.BlockSpec((1, Q), lambda i: (0, 0)),
            pl.BlockSpec((1, Q), lambda i: (0, 0)),
            pl.BlockSpec((1, Q), lambda i: (0, 0)),
            pl.BlockSpec((Q, 128), lambda i: (0, 0)),
            pl.BlockSpec((1, 128), lambda i: (0, 0)),
        ],
        out_specs=[
            pl.BlockSpec((BM1, K), lambda i: (i, 0)),
            pl.BlockSpec((BM1, 128), lambda i: (i, 0)),
            pl.BlockSpec((BM1, 1), lambda i: (i, 0)),
            pl.BlockSpec((K, Q), lambda i: (0, 0)),
        ],
        out_shape=[
            jax.ShapeDtypeStruct((N, K), jnp.float32),
            jax.ShapeDtypeStruct((N, 128), jnp.bfloat16),
            jax.ShapeDtypeStruct((N, 1), jnp.int32),
            jax.ShapeDtypeStruct((K, Q), jnp.float32),
        ],
        scratch_shapes=[
            pltpu.VMEM((1, 128), jnp.float32),
            pltpu.VMEM((128, 128), jnp.float32),
        ],
    )(Z, wtT, bt, lnw, lnb, woT, bo)

    pt = p_bf[:, :K].astype(jnp.float32).T.reshape(-1)

    b_sc = _sc_call(A, pt).reshape(SC_ROWS, K * 16)

    at_tc = pl.pallas_call(
        _stage2_body,
        grid=(G2,),
        in_specs=[
            pl.BlockSpec((BM2, N), lambda i: (i, 0)),
            pl.BlockSpec((N, 128), lambda i: (0, 0)),
            pl.BlockSpec((BM2, 128), lambda i: (i, 0)),
        ],
        out_specs=pl.BlockSpec((128, 128), lambda i: (0, 0)),
        out_shape=jax.ShapeDtypeStruct((128, 128), jnp.float32),
        scratch_shapes=[pltpu.VMEM((128, 128), jnp.float32)],
    )(A, p_bf, p_bf)

    fold = (lax.broadcasted_iota(jnp.int32, (K * 16, 128), 0) // 16
            == lax.broadcasted_iota(jnp.int32, (K * 16, 128), 1)
            ).astype(jnp.float32)

    A_tilde = pl.pallas_call(
        _stage2b_body,
        grid=(1,),
        in_specs=[
            pl.BlockSpec((SC_ROWS, K * 16), lambda i: (0, 0)),
            pl.BlockSpec((K * 16, 128), lambda i: (0, 0)),
            pl.BlockSpec((SC_ROWS, N - SC_COLS), lambda i: (0, 0)),
            pl.BlockSpec((N - SC_COLS, 128), lambda i: (0, 0)),
            pl.BlockSpec((SC_ROWS, 128), lambda i: (0, 0)),
            pl.BlockSpec((128, 128), lambda i: (0, 0)),
        ],
        out_specs=pl.BlockSpec((K, K), lambda i: (0, 0)),
        out_shape=jax.ShapeDtypeStruct((K, K), jnp.float32),
    )(b_sc, fold, A[TC_ROWS:, SC_COLS:],
      p_bf[SC_COLS:].astype(jnp.float32), p_bf[TC_ROWS:].astype(jnp.float32),
      at_tc)

    S = s2d[:, 0]
    return X_tilde, A_tilde, P, S


# speedup vs baseline: 1.8533x; 1.1143x over previous
import functools

import jax
import jax.numpy as jnp
from jax import lax
from jax.experimental import pallas as pl
from jax.experimental.pallas import tpu as pltpu
from jax.experimental.pallas import tpu_sc as plsc

N, Q, K = 10000, 128, 10
BM1 = 2000

BMM = 200
RB = 4
UNITS = N // BMM

NW = 32
NCHK = N // 16
CPT = 20


def _stage1_body(z_ref, wtT_ref, bt_ref, lnw_ref, lnb_ref, woT_ref, bo_ref,
                 p_ref, pbf_ref, x_ref, colsum_ref, ztp_ref):
    step = pl.program_id(0)
    nsteps = pl.num_programs(0)

    z = z_ref[...]
    m = jnp.dot(z, wtT_ref[...], preferred_element_type=jnp.float32)
    m = m + bt_ref[...]
    mu = jnp.mean(m, axis=1, keepdims=True)
    var = jnp.mean((m - mu) * (m - mu), axis=1, keepdims=True)
    mn = (m - mu) / jnp.sqrt(var + 1e-5) * lnw_ref[...] + lnb_ref[...]
    h = jnp.where(mn >= 0, mn, 0.2 * mn)
    ol = jnp.dot(h, woT_ref[...], preferred_element_type=jnp.float32)
    ol = ol + bo_ref[...]
    olmax = jnp.max(ol, axis=1, keepdims=True)
    e = jnp.exp(ol - olmax)
    p = e / jnp.sum(e, axis=1, keepdims=True)
    p_ref[...] = p[:, :K]
    pbf_ref[...] = p.astype(jnp.bfloat16)

    @pl.when(step == 0)
    def _init():
        colsum_ref[...] = jnp.zeros_like(colsum_ref)
        ztp_ref[...] = jnp.zeros_like(ztp_ref)

    colsum_ref[...] += jnp.sum(p, axis=0, keepdims=True)
    ztp_ref[...] += lax.dot_general(z, p, (((0,), (0,)), ((), ())),
                                    preferred_element_type=jnp.float32)

    @pl.when(step == nsteps - 1)
    def _fin():
        cs = colsum_ref[...]
        lane1 = lax.broadcasted_iota(jnp.int32, cs.shape, 1)
        d = jnp.where(lane1 < K, 1.0 / cs + 1e-8, 0.0)
        x_ref[...] = (ztp_ref[...] * d).T[:K, :]


def _sc_argmax_body(pt_hbm, s_hbm, pcl, sbuf, semp, semo):
    sid = lax.axis_index("s")
    cid = lax.axis_index("c")
    wid = sid * 2 + cid
    nch = jnp.where(wid < 17, CPT, CPT - 1)
    chunk0 = wid * (CPT - 1) + jnp.minimum(wid, 17)
    row0 = chunk0 * 16

    for c in range(K):
        pltpu.async_copy(
            pt_hbm.at[pl.ds(c * N + row0, CPT * 16)],
            pcl.at[pl.ds(c * CPT * 16, CPT * 16)], semp).wait()

    def body(jj, _):
        off = jj * 16
        best_v = pcl[pl.ds(off, 16)]
        best_i = jnp.zeros((16,), jnp.int32)
        for c in range(1, K):
            v = pcl[pl.ds(c * CPT * 16 + off, 16)]
            upd = v > best_v
            best_i = jnp.where(upd, c, best_i)
            best_v = jnp.where(upd, v, best_v)
        sbuf[pl.ds(off, 16)] = best_i
        return 0

    lax.fori_loop(0, nch, body, 0)

    @pl.when(wid < 17)
    def _():
        cp = pltpu.make_async_copy(
            sbuf.at[pl.ds(0, CPT * 16)],
            s_hbm.at[pl.ds(row0, CPT * 16)], semo)
        cp.start()
        cp.wait()

    @pl.when(wid >= 17)
    def _():
        cp = pltpu.make_async_copy(
            sbuf.at[pl.ds(0, (CPT - 1) * 16)],
            s_hbm.at[pl.ds(row0, (CPT - 1) * 16)], semo)
        cp.start()
        cp.wait()


_sc_argmax = functools.partial(
    pl.kernel,
    mesh=plsc.VectorSubcoreMesh(core_axis_name="c", subcore_axis_name="s"),
    out_type=jax.ShapeDtypeStruct((N,), jnp.int32),
    scratch_types=[
        pltpu.VMEM((K * CPT * 16,), jnp.float32),
        pltpu.VMEM((CPT * 16,), jnp.int32),
        pltpu.SemaphoreType.DMA,
        pltpu.SemaphoreType.DMA,
    ],
)(_sc_argmax_body)


def _stage2_body(pbf_ref, a_hbm, at_ref, ab0, ab1, ab2, ab3,
                 s0, s1, s2, s3):
    abufs = (ab0, ab1, ab2, ab3)
    sems = (s0, s1, s2, s3)

    def cp(g, u):
        rs = pl.multiple_of(g * BMM, 8)
        return pltpu.make_async_copy(
            a_hbm.at[pl.ds(rs, BMM), :], abufs[u], sems[u])

    for u in range(RB):
        cp(u, u).start()
    at_ref[...] = jnp.zeros_like(at_ref)
    pbf = pbf_ref[...]

    def unit(g, u):
        cp(g, u).wait()
        a_bf = abufs[u][...].astype(jnp.bfloat16)
        ap = jnp.dot(a_bf, pbf, preferred_element_type=jnp.float32)
        rs = pl.multiple_of(g * BMM, 8)
        pblk = pbf_ref[pl.ds(rs, BMM), :].astype(jnp.float32)
        at_ref[...] += lax.dot_general(pblk, ap, (((0,), (0,)), ((), ())),
                                       preferred_element_type=jnp.float32)

        @pl.when(g + RB < UNITS)
        def _():
            cp(g + RB, u).start()

    def loop_body(t, _):
        for u in range(RB):
            unit(t * RB + u, u)
        return 0

    lax.fori_loop(0, UNITS // RB, loop_body, 0)
    for i, u in enumerate(range(UNITS - UNITS % RB, UNITS)):
        unit(u, i)


def kernel(Z, A, W_t, b_t, ln_w, ln_b, W_o, b_o):
    wtT = W_t.T
    bt = b_t.reshape(1, Q)
    lnw = ln_w.reshape(1, Q)
    lnb = ln_b.reshape(1, Q)
    woT = jnp.zeros((Q, 128), jnp.float32).at[:, :K].set(W_o.T)
    bo = jnp.full((1, 128), -1e30, jnp.float32).at[0, :K].set(b_o)

    grid1 = N // BM1
    P, p_bf, X_tilde = pl.pallas_call(
        _stage1_body,
        grid=(grid1,),
        in_specs=[
            pl.BlockSpec((BM1, Q), lambda i: (i, 0)),
            pl.BlockSpec((Q, Q), lambda i: (0, 0)),
            pl.BlockSpec((1, Q), lambda i: (0, 0)),
            pl.BlockSpec((1, Q), lambda i: (0, 0)),
            pl.BlockSpec((1, Q), lambda i: (0, 0)),
            pl.BlockSpec((Q, 128), lambda i: (0, 0)),
            pl.BlockSpec((1, 128), lambda i: (0, 0)),
        ],
        out_specs=[
            pl.BlockSpec((BM1, K), lambda i: (i, 0)),
            pl.BlockSpec((BM1, 128), lambda i: (i, 0)),
            pl.BlockSpec((K, Q), lambda i: (0, 0)),
        ],
        out_shape=[
            jax.ShapeDtypeStruct((N, K), jnp.float32),
            jax.ShapeDtypeStruct((N, 128), jnp.bfloat16),
            jax.ShapeDtypeStruct((K, Q), jnp.float32),
        ],
        scratch_shapes=[
            pltpu.VMEM((1, 128), jnp.float32),
            pltpu.VMEM((128, 128), jnp.float32),
        ],
    )(Z, wtT, bt, lnw, lnb, woT, bo)

    pt32 = jnp.pad(P.T.reshape(-1), (0, 128))

    S = _sc_argmax(pt32)

    at_full = pl.pallas_call(
        _stage2_body,
        in_specs=[
            pl.BlockSpec((N, 128), lambda: (0, 0)),
            pl.BlockSpec(memory_space=pl.ANY),
        ],
        out_specs=pl.BlockSpec((128, 128), lambda: (0, 0)),
        out_shape=jax.ShapeDtypeStruct((128, 128), jnp.float32),
        scratch_shapes=[
            pltpu.VMEM((BMM, N), jnp.float32),
            pltpu.VMEM((BMM, N), jnp.float32),
            pltpu.VMEM((BMM, N), jnp.float32),
            pltpu.VMEM((BMM, N), jnp.float32),
            pltpu.SemaphoreType.DMA,
            pltpu.SemaphoreType.DMA,
            pltpu.SemaphoreType.DMA,
            pltpu.SemaphoreType.DMA,
        ],
    )(p_bf, A)

    A_tilde = at_full[:K, :K]
    return X_tilde, A_tilde, P, S
